# baseline (speedup 1.0000x reference)
import functools

import jax
import jax.numpy as jnp
from jax import lax
from jax.experimental import pallas as pl
from jax.experimental.pallas import tpu as pltpu
from jax.experimental.pallas import tpu_sc as plsc

N_NODES = 10000
N_EDGES = 320000
D_FEAT = 128

NC = 2
NS = 16
HALF = N_NODES // NC
CHUNK = 48
SLAB = 8
K = 424
KS = K // SLAB
E_PAD = NS * K * CHUNK
X_ROWS_PER_TILE = 632
X_PAD = NS * X_ROWS_PER_TILE
A_ROWS_PER_TILE = 320
A_PAD = NS * A_ROWS_PER_TILE
N_DUMMY = A_PAD - HALF


def _sc_scatter(xpad, src_p, dst_p, zer):
    mesh = plsc.VectorSubcoreMesh(
        core_axis_name="c", subcore_axis_name="s", num_cores=NC, num_subcores=NS
    )

    @functools.partial(
        pl.kernel,
        out_type=jax.ShapeDtypeStruct((NC, A_PAD, D_FEAT), jnp.float32),
        mesh=mesh,
        scratch_types=[
            pltpu.VMEM((SLAB, CHUNK), jnp.int32),
            pltpu.VMEM((SLAB, CHUNK), jnp.int32),
            pltpu.VMEM((CHUNK, D_FEAT), jnp.float32),
            pltpu.VMEM_SHARED((X_PAD, D_FEAT), jnp.float32),
            pltpu.VMEM_SHARED((A_PAD, D_FEAT), jnp.float32),
            pltpu.SemaphoreType.DMA,
        ],
    )
    def k(x_hbm, src_hbm, dst_hbm, zer_hbm, out_hbm, src_v, dst_v, rows_v,
          x_sp, acc, sa):
        cid = lax.axis_index("c")
        sid = lax.axis_index("s")
        xbase = sid * X_ROWS_PER_TILE
        abase = sid * A_ROWS_PER_TILE
        pltpu.sync_copy(x_hbm.at[pl.ds(xbase, X_ROWS_PER_TILE)],
                        x_sp.at[pl.ds(xbase, X_ROWS_PER_TILE)])
        pltpu.sync_copy(zer_hbm, acc.at[pl.ds(abase, A_ROWS_PER_TILE)])
        plsc.subcore_barrier()

        def slab_step(t, carry):
            pltpu.sync_copy(src_hbm.at[sid, t], src_v)
            pltpu.sync_copy(dst_hbm.at[cid, sid, t], dst_v)
            for b in range(SLAB):
                pltpu.async_copy(x_sp.at[src_v.at[b]], rows_v, sa).wait()
                pltpu.sync_copy(rows_v, acc.at[dst_v.at[b]], add=True)
            return carry

        lax.fori_loop(0, KS, slab_step, 0)
        plsc.subcore_barrier()
        pltpu.sync_copy(
            acc.at[pl.ds(abase, A_ROWS_PER_TILE)],
            out_hbm.at[cid, pl.ds(abase, A_ROWS_PER_TILE)],
        )

    return k(xpad, src_p, dst_p, zer)


def kernel(x, edge_index):
    src = edge_index[0].astype(jnp.int32)
    dst = edge_index[1].astype(jnp.int32)
    pad = E_PAD - N_EDGES
    pad_ar = jnp.arange(pad, dtype=jnp.int32)
    src_p = jnp.concatenate([src, pad_ar % N_NODES])
    dst_full = jnp.concatenate([dst, jnp.full((pad,), -1, jnp.int32)])
    e_ar = jnp.arange(E_PAD, dtype=jnp.int32)
    dummy = HALF + (e_ar % N_DUMMY)
    halves = []
    for h in range(NC):
        owned = (dst_full >= h * HALF) & (dst_full < (h + 1) * HALF)
        halves.append(jnp.where(owned, dst_full - h * HALF, dummy))
    dst_p = jnp.stack(halves).reshape(NC, NS, KS, SLAB, CHUNK)
    src_p = src_p.reshape(NS, KS, SLAB, CHUNK)
    xpad = jnp.concatenate([x, jnp.zeros((X_PAD - N_NODES, D_FEAT), jnp.float32)])
    zer = jnp.zeros((A_ROWS_PER_TILE, D_FEAT), jnp.float32)
    partials = _sc_scatter(xpad, src_p, dst_p, zer)
    return jnp.concatenate([partials[0, :HALF], partials[1, :HALF]])


# speedup vs baseline: 2.7238x; 2.7238x over previous
import functools

import jax
import jax.numpy as jnp
from jax import lax
from jax.experimental import pallas as pl
from jax.experimental.pallas import tpu as pltpu
from jax.experimental.pallas import tpu_sc as plsc

N_NODES = 10000
N_EDGES = 320000
D_FEAT = 128

NC = 2
NS = 16
CHUNK = 128
SLAB = 16
K = 80
KS = K // SLAB
E_PAD = NC * NS * K * CHUNK
ROWS_PER_TILE = 632
N_PAD = NS * ROWS_PER_TILE
N_DUMMY = N_PAD - N_NODES


def _sc_scatter(x, src_p, dst_p, zer):
    mesh = plsc.VectorSubcoreMesh(
        core_axis_name="c", subcore_axis_name="s", num_cores=NC, num_subcores=NS
    )

    @functools.partial(
        pl.kernel,
        out_type=jax.ShapeDtypeStruct((NC, N_PAD, D_FEAT), jnp.float32),
        mesh=mesh,
        scratch_types=[
            pltpu.VMEM((SLAB, CHUNK), jnp.int32),
            pltpu.VMEM((SLAB, CHUNK), jnp.int32),
            pltpu.VMEM((CHUNK, D_FEAT), jnp.float32),
            pltpu.VMEM((CHUNK, D_FEAT), jnp.float32),
            pltpu.VMEM_SHARED((N_PAD, D_FEAT), jnp.float32),
            pltpu.SemaphoreType.DMA,
            pltpu.SemaphoreType.DMA,
            pltpu.SemaphoreType.DMA,
        ],
    )
    def k(x_hbm, src_hbm, dst_hbm, zer_hbm, out_hbm,
          src_v, dst_v, ra, rb, acc, sg, ssa, ssb):
        cid = lax.axis_index("c")
        sid = lax.axis_index("s")
        rbase = sid * ROWS_PER_TILE
        pltpu.sync_copy(zer_hbm, acc.at[pl.ds(rbase, ROWS_PER_TILE)])
        plsc.subcore_barrier()

        def body(t, carry):
            r = t & 7
            slab = t >> 3

            @pl.when((r == 0) & (t > 0))
            def _():
                pltpu.make_async_copy(ra, acc.at[dst_v.at[0]], ssa).wait()
                pltpu.make_async_copy(rb, acc.at[dst_v.at[1]], ssb).wait()

            @pl.when(r == 0)
            def _():
                pltpu.sync_copy(src_hbm.at[cid, sid, slab], src_v)
                pltpu.sync_copy(dst_hbm.at[cid, sid, slab], dst_v)

            @pl.when(r != 0)
            def _():
                pltpu.make_async_copy(ra, acc.at[dst_v.at[0]], ssa).wait()

            pltpu.async_copy(x_hbm.at[src_v.at[2 * r]], ra, sg).wait()
            pltpu.async_copy(ra, acc.at[dst_v.at[2 * r]], ssa, add=True)

            @pl.when(r != 0)
            def _():
                pltpu.make_async_copy(rb, acc.at[dst_v.at[1]], ssb).wait()

            pltpu.async_copy(x_hbm.at[src_v.at[2 * r + 1]], rb, sg).wait()
            pltpu.async_copy(rb, acc.at[dst_v.at[2 * r + 1]], ssb, add=True)
            return carry

        lax.fori_loop(0, K // 2, body, 0)
        pltpu.make_async_copy(ra, acc.at[dst_v.at[0]], ssa).wait()
        pltpu.make_async_copy(rb, acc.at[dst_v.at[1]], ssb).wait()
        plsc.subcore_barrier()
        pltpu.sync_copy(
            acc.at[pl.ds(rbase, ROWS_PER_TILE)],
            out_hbm.at[cid, pl.ds(rbase, ROWS_PER_TILE)],
        )

    return k(x, src_p, dst_p, zer)


def _combine(p):
    blk = 2528

    def body(a_ref, b_ref, o_ref):
        o_ref[...] = a_ref[0] + b_ref[0]

    return pl.pallas_call(
        body,
        grid=(N_PAD // blk,),
        in_specs=[
            pl.BlockSpec((1, blk, D_FEAT), lambda i: (0, i, 0)),
            pl.BlockSpec((1, blk, D_FEAT), lambda i: (1, i, 0)),
        ],
        out_specs=pl.BlockSpec((blk, D_FEAT), lambda i: (i, 0)),
        out_shape=jax.ShapeDtypeStruct((N_PAD, D_FEAT), jnp.float32),
    )(p, p)


def kernel(x, edge_index):
    src = edge_index[0].astype(jnp.int32)
    dst = edge_index[1].astype(jnp.int32)
    pad = E_PAD - N_EDGES
    pad_ar = jnp.arange(pad, dtype=jnp.int32)
    src_p = jnp.concatenate([src, pad_ar % N_NODES])
    dst_p = jnp.concatenate([dst, N_NODES + (pad_ar % N_DUMMY)])
    src_p = src_p.reshape(NC, NS, KS, SLAB, CHUNK)
    dst_p = dst_p.reshape(NC, NS, KS, SLAB, CHUNK)
    zer = jnp.zeros((ROWS_PER_TILE, D_FEAT), jnp.float32)
    partials = _sc_scatter(x, src_p, dst_p, zer)
    out = _combine(partials)
    return out[:N_NODES]


# speedup vs baseline: 2.7850x; 1.0225x over previous
import functools

import jax
import jax.numpy as jnp
from jax import lax
from jax.experimental import pallas as pl
from jax.experimental.pallas import tpu as pltpu
from jax.experimental.pallas import tpu_sc as plsc

N_NODES = 10000
N_EDGES = 320000
D_FEAT = 128

NC = 2
NS = 16
CHUNK = 128
SLAB = 20
K = 80
KS = K // SLAB
E_PAD = NC * NS * K * CHUNK
ROWS_PER_TILE = 632
N_PAD = NS * ROWS_PER_TILE
N_DUMMY = N_PAD - N_NODES


def _sc_scatter(x, src_p, dst_p, zer):
    mesh = plsc.VectorSubcoreMesh(
        core_axis_name="c", subcore_axis_name="s", num_cores=NC, num_subcores=NS
    )

    @functools.partial(
        pl.kernel,
        out_type=jax.ShapeDtypeStruct((NC, N_PAD, D_FEAT), jnp.float32),
        mesh=mesh,
        scratch_types=[
            pltpu.VMEM((SLAB, CHUNK), jnp.int32),
            pltpu.VMEM((SLAB, CHUNK), jnp.int32),
            pltpu.VMEM((CHUNK, D_FEAT), jnp.float32),
            pltpu.VMEM((CHUNK, D_FEAT), jnp.float32),
            pltpu.VMEM_SHARED((N_PAD, D_FEAT), jnp.float32),
            pltpu.SemaphoreType.DMA,
            pltpu.SemaphoreType.DMA,
            pltpu.SemaphoreType.DMA,
        ],
    )
    def k(x_hbm, src_hbm, dst_hbm, zer_hbm, out_hbm,
          src_v, dst_v, ra, rb, acc, sg, ssa, ssb):
        cid = lax.axis_index("c")
        sid = lax.axis_index("s")
        rbase = sid * ROWS_PER_TILE
        half = SLAB // 2
        pltpu.sync_copy(src_hbm.at[cid, sid, 0], src_v)
        pltpu.sync_copy(dst_hbm.at[cid, sid, 0], dst_v)
        pltpu.sync_copy(zer_hbm, acc.at[pl.ds(rbase, ROWS_PER_TILE)])
        plsc.subcore_barrier()

        def body(t, carry):
            r = t % half
            slab = t // half

            @pl.when((r == 0) & (t > 0))
            def _():
                pltpu.make_async_copy(ra, acc.at[dst_v.at[0]], ssa).wait()
                pltpu.make_async_copy(rb, acc.at[dst_v.at[1]], ssb).wait()
                pltpu.sync_copy(src_hbm.at[cid, sid, slab], src_v)
                pltpu.sync_copy(dst_hbm.at[cid, sid, slab], dst_v)

            @pl.when(r != 0)
            def _():
                pltpu.make_async_copy(ra, acc.at[dst_v.at[0]], ssa).wait()

            pltpu.async_copy(x_hbm.at[src_v.at[2 * r]], ra, sg).wait()
            pltpu.async_copy(ra, acc.at[dst_v.at[2 * r]], ssa, add=True)

            @pl.when(r != 0)
            def _():
                pltpu.make_async_copy(rb, acc.at[dst_v.at[1]], ssb).wait()

            pltpu.async_copy(x_hbm.at[src_v.at[2 * r + 1]], rb, sg).wait()
            pltpu.async_copy(rb, acc.at[dst_v.at[2 * r + 1]], ssb, add=True)
            return carry

        lax.fori_loop(0, K // 2, body, 0)
        pltpu.make_async_copy(ra, acc.at[dst_v.at[0]], ssa).wait()
        pltpu.make_async_copy(rb, acc.at[dst_v.at[1]], ssb).wait()
        plsc.subcore_barrier()
        pltpu.sync_copy(
            acc.at[pl.ds(rbase, ROWS_PER_TILE)],
            out_hbm.at[cid, pl.ds(rbase, ROWS_PER_TILE)],
        )

    return k(x, src_p, dst_p, zer)


def _combine(p):
    blk = 2000

    def body(a_ref, b_ref, o_ref):
        o_ref[...] = a_ref[0] + b_ref[0]

    return pl.pallas_call(
        body,
        grid=(N_NODES // blk,),
        in_specs=[
            pl.BlockSpec((1, blk, D_FEAT), lambda i: (0, i, 0)),
            pl.BlockSpec((1, blk, D_FEAT), lambda i: (1, i, 0)),
        ],
        out_specs=pl.BlockSpec((blk, D_FEAT), lambda i: (i, 0)),
        out_shape=jax.ShapeDtypeStruct((N_NODES, D_FEAT), jnp.float32),
    )(p, p)


def kernel(x, edge_index):
    src = edge_index[0].astype(jnp.int32)
    dst = edge_index[1].astype(jnp.int32)
    pad = E_PAD - N_EDGES
    pad_ar = jnp.arange(pad, dtype=jnp.int32)
    src_p = jnp.concatenate([src, pad_ar % N_NODES])
    dst_p = jnp.concatenate([dst, N_NODES + (pad_ar % N_DUMMY)])
    src_p = src_p.reshape(NC, NS, KS, SLAB, CHUNK)
    dst_p = dst_p.reshape(NC, NS, KS, SLAB, CHUNK)
    zer = jnp.zeros((ROWS_PER_TILE, D_FEAT), jnp.float32)
    partials = _sc_scatter(x, src_p, dst_p, zer)
    return _combine(partials)


# speedup vs baseline: 2.8543x; 1.0249x over previous
import functools

import jax
import jax.numpy as jnp
from jax import lax
from jax.experimental import pallas as pl
from jax.experimental.pallas import tpu as pltpu
from jax.experimental.pallas import tpu_sc as plsc

N_NODES = 10000
N_EDGES = 320000
D_FEAT = 128

NC = 2
NS = 16
CHUNK = 128
SLAB = 40
K = 80
KS = K // SLAB
E_PAD = NC * NS * K * CHUNK
ROWS_PER_TILE = 632
N_PAD = NS * ROWS_PER_TILE
N_DUMMY = N_PAD - N_NODES


def _sc_scatter(x, src_p, dst_p, zer):
    mesh = plsc.VectorSubcoreMesh(
        core_axis_name="c", subcore_axis_name="s", num_cores=NC, num_subcores=NS
    )

    @functools.partial(
        pl.kernel,
        out_type=jax.ShapeDtypeStruct((NC, N_PAD, D_FEAT), jnp.float32),
        mesh=mesh,
        scratch_types=[
            pltpu.VMEM((SLAB, CHUNK), jnp.int32),
            pltpu.VMEM((SLAB, CHUNK), jnp.int32),
            pltpu.VMEM((CHUNK, D_FEAT), jnp.float32),
            pltpu.VMEM((CHUNK, D_FEAT), jnp.float32),
            pltpu.VMEM_SHARED((N_PAD, D_FEAT), jnp.float32),
            pltpu.SemaphoreType.DMA,
            pltpu.SemaphoreType.DMA,
            pltpu.SemaphoreType.DMA,
        ],
    )
    def k(x_hbm, src_hbm, dst_hbm, zer_hbm, out_hbm,
          src_v, dst_v, ra, rb, acc, sg, ssa, ssb):
        cid = lax.axis_index("c")
        sid = lax.axis_index("s")
        rbase = sid * ROWS_PER_TILE
        half = SLAB // 2
        pltpu.sync_copy(src_hbm.at[cid, sid, 0], src_v)
        pltpu.sync_copy(dst_hbm.at[cid, sid, 0], dst_v)
        pltpu.sync_copy(zer_hbm, acc.at[pl.ds(rbase, ROWS_PER_TILE)])
        plsc.subcore_barrier()

        def body(t, carry):
            r = t % half
            slab = t // half

            @pl.when((r == 0) & (t > 0))
            def _():
                pltpu.make_async_copy(ra, acc.at[dst_v.at[0]], ssa).wait()
                pltpu.make_async_copy(rb, acc.at[dst_v.at[1]], ssb).wait()
                pltpu.sync_copy(src_hbm.at[cid, sid, slab], src_v)
                pltpu.sync_copy(dst_hbm.at[cid, sid, slab], dst_v)

            @pl.when(r != 0)
            def _():
                pltpu.make_async_copy(ra, acc.at[dst_v.at[0]], ssa).wait()

            pltpu.async_copy(x_hbm.at[src_v.at[2 * r]], ra, sg).wait()
            pltpu.async_copy(ra, acc.at[dst_v.at[2 * r]], ssa, add=True)

            @pl.when(r != 0)
            def _():
                pltpu.make_async_copy(rb, acc.at[dst_v.at[1]], ssb).wait()

            pltpu.async_copy(x_hbm.at[src_v.at[2 * r + 1]], rb, sg).wait()
            pltpu.async_copy(rb, acc.at[dst_v.at[2 * r + 1]], ssb, add=True)
            return carry

        lax.fori_loop(0, K // 2, body, 0)
        pltpu.make_async_copy(ra, acc.at[dst_v.at[0]], ssa).wait()
        pltpu.make_async_copy(rb, acc.at[dst_v.at[1]], ssb).wait()
        plsc.subcore_barrier()
        pltpu.sync_copy(
            acc.at[pl.ds(rbase, ROWS_PER_TILE)],
            out_hbm.at[cid, pl.ds(rbase, ROWS_PER_TILE)],
        )

    return k(x, src_p, dst_p, zer)


def _combine(p):
    blk = 2000

    def body(a_ref, b_ref, o_ref):
        o_ref[...] = a_ref[0] + b_ref[0]

    return pl.pallas_call(
        body,
        grid=(N_NODES // blk,),
        in_specs=[
            pl.BlockSpec((1, blk, D_FEAT), lambda i: (0, i, 0)),
            pl.BlockSpec((1, blk, D_FEAT), lambda i: (1, i, 0)),
        ],
        out_specs=pl.BlockSpec((blk, D_FEAT), lambda i: (i, 0)),
        out_shape=jax.ShapeDtypeStruct((N_NODES, D_FEAT), jnp.float32),
    )(p, p)


def kernel(x, edge_index):
    src = edge_index[0].astype(jnp.int32)
    dst = edge_index[1].astype(jnp.int32)
    pad = E_PAD - N_EDGES
    pad_ar = jnp.arange(pad, dtype=jnp.int32)
    src_p = jnp.concatenate([src, pad_ar % N_NODES])
    dst_p = jnp.concatenate([dst, N_NODES + (pad_ar % N_DUMMY)])
    src_p = src_p.reshape(NC, NS, KS, SLAB, CHUNK)
    dst_p = dst_p.reshape(NC, NS, KS, SLAB, CHUNK)
    zer = jnp.zeros((ROWS_PER_TILE, D_FEAT), jnp.float32)
    partials = _sc_scatter(x, src_p, dst_p, zer)
    return _combine(partials)
